# Initial kernel scaffold; baseline (speedup 1.0000x reference)
import jax
import jax.numpy as jnp
from jax.experimental import pallas as pl


def kernel(x, w1, b1, gamma1, beta1, w2, b2, gamma2, beta2):
    raise NotImplementedError("write your pallas kernel here")


# speedup vs baseline: 1.5151x; 1.5151x over previous
import jax
import jax.numpy as jnp
from jax.experimental import pallas as pl
from jax.experimental.pallas import tpu as pltpu


def _conv_relu_stats_kernel(H, W, Cin, Cout, K):
    PAD = K // 2
    HW = H * W
    Hp = H + 2 * PAD

    def body(x_ref, w_ref, b_ref, y_ref, st_ref, xpad_ref, p_ref):
        xpad_ref[...] = jnp.zeros_like(xpad_ref)
        xpad_ref[PAD:PAD + H, PAD:PAD + W, :] = x_ref[0]
        for kw in range(K):
            p_ref[:, kw * Cin:(kw + 1) * Cin] = (
                xpad_ref[:, kw:kw + W, :].reshape(Hp * W, Cin))
        acc = jnp.zeros((HW, Cout), jnp.float32)
        for kh in range(K):
            acc += jnp.dot(p_ref[kh * W:kh * W + HW, :], w_ref[kh],
                           preferred_element_type=jnp.float32)
        acc = jnp.maximum(acc + b_ref[...], 0.0)
        y_ref[0] = acc.astype(jnp.bfloat16)
        st_ref[0] = jnp.concatenate(
            [jnp.sum(acc, axis=0, keepdims=True),
             jnp.sum(acc * acc, axis=0, keepdims=True)], axis=0)

    return body


def _bn_conv_relu_pool_stats_kernel(H, W, C, K):
    PAD = K // 2
    HW = H * W
    Hp = H + 2 * PAD
    Ho, Wo = H // 2, W // 2

    def body(y1_ref, sc_ref, sh_ref, w_ref, b_ref, y2_ref, st_ref,
             xpad_ref, p_ref):
        z = (y1_ref[0].astype(jnp.float32) * sc_ref[...] + sh_ref[...]
             ).astype(jnp.bfloat16)
        xpad_ref[...] = jnp.zeros_like(xpad_ref)
        xpad_ref[PAD:PAD + H, PAD:PAD + W, :] = z.reshape(H, W, C)
        for kw in range(K):
            p_ref[:, kw * C:(kw + 1) * C] = (
                xpad_ref[:, kw:kw + W, :].reshape(Hp * W, C))
        acc = jnp.zeros((HW, C), jnp.float32)
        for kh in range(K):
            acc += jnp.dot(p_ref[kh * W:kh * W + HW, :], w_ref[kh],
                           preferred_element_type=jnp.float32)
        acc = jnp.maximum(acc + b_ref[...], 0.0)
        a = acc.reshape(Ho, 2, W, C)
        ph = jnp.maximum(a[:, 0], a[:, 1])
        pw = ph.reshape(Ho, Wo, 2, C)
        pooled = jnp.maximum(pw[:, :, 0], pw[:, :, 1]).reshape(Ho * Wo, C)
        y2_ref[0] = pooled.astype(jnp.bfloat16)
        st_ref[0] = jnp.concatenate(
            [jnp.sum(pooled, axis=0, keepdims=True),
             jnp.sum(pooled * pooled, axis=0, keepdims=True)], axis=0)

    return body


def _affine_kernel(y_ref, sc_ref, sh_ref, o_ref):
    o_ref[...] = y_ref[...].astype(jnp.float32) * sc_ref[...] + sh_ref[...]


def _bn_scale_shift(stats, count, gamma, beta, eps=1e-5):
    s = jnp.sum(stats, axis=0)
    mean = s[0] / count
    var = jnp.maximum(s[1] / count - mean * mean, 0.0)
    scale = gamma * jax.lax.rsqrt(var + eps)
    shift = beta - mean * scale
    return (scale.reshape(1, -1).astype(jnp.float32),
            shift.reshape(1, -1).astype(jnp.float32))


def kernel(x, w1, b1, gamma1, beta1, w2, b2, gamma2, beta2):
    N, Cin, H, W = x.shape
    K = w1.shape[0]
    Cout = w1.shape[-1]
    PAD = K // 2
    Hp, Wp = H + 2 * PAD, W + 2 * PAD
    Ho, Wo = H // 2, W // 2

    xh = jnp.transpose(x, (0, 2, 3, 1)).astype(jnp.bfloat16)
    w1r = w1.reshape(K, K * Cin, Cout).astype(jnp.bfloat16)
    w2r = w2.reshape(K, K * Cout, Cout).astype(jnp.bfloat16)
    b1r = b1.reshape(1, Cout).astype(jnp.float32)
    b2r = b2.reshape(1, Cout).astype(jnp.float32)

    parallel = pltpu.CompilerParams(dimension_semantics=("parallel",))

    y1, st1 = pl.pallas_call(
        _conv_relu_stats_kernel(H, W, Cin, Cout, K),
        out_shape=(jax.ShapeDtypeStruct((N, H * W, Cout), jnp.bfloat16),
                   jax.ShapeDtypeStruct((N, 2, Cout), jnp.float32)),
        grid=(N,),
        in_specs=[pl.BlockSpec((1, H, W, Cin), lambda n: (n, 0, 0, 0)),
                  pl.BlockSpec((K, K * Cin, Cout), lambda n: (0, 0, 0)),
                  pl.BlockSpec((1, Cout), lambda n: (0, 0))],
        out_specs=(pl.BlockSpec((1, H * W, Cout), lambda n: (n, 0, 0)),
                   pl.BlockSpec((1, 2, Cout), lambda n: (n, 0, 0))),
        scratch_shapes=[pltpu.VMEM((Hp, Wp, Cin), jnp.bfloat16),
                        pltpu.VMEM((Hp * W, K * Cin), jnp.bfloat16)],
        compiler_params=parallel,
    )(xh, w1r, b1r)
    sc1, sh1 = _bn_scale_shift(st1, N * H * W, gamma1, beta1)

    y2, st2 = pl.pallas_call(
        _bn_conv_relu_pool_stats_kernel(H, W, Cout, K),
        out_shape=(jax.ShapeDtypeStruct((N, Ho * Wo, Cout), jnp.bfloat16),
                   jax.ShapeDtypeStruct((N, 2, Cout), jnp.float32)),
        grid=(N,),
        in_specs=[pl.BlockSpec((1, H * W, Cout), lambda n: (n, 0, 0)),
                  pl.BlockSpec((1, Cout), lambda n: (0, 0)),
                  pl.BlockSpec((1, Cout), lambda n: (0, 0)),
                  pl.BlockSpec((K, K * Cout, Cout), lambda n: (0, 0, 0)),
                  pl.BlockSpec((1, Cout), lambda n: (0, 0))],
        out_specs=(pl.BlockSpec((1, Ho * Wo, Cout), lambda n: (n, 0, 0)),
                   pl.BlockSpec((1, 2, Cout), lambda n: (n, 0, 0))),
        scratch_shapes=[pltpu.VMEM((Hp, Wp, Cout), jnp.bfloat16),
                        pltpu.VMEM((Hp * W, K * Cout), jnp.bfloat16)],
        compiler_params=parallel,
    )(y1, sc1, sh1, w2r, b2r)
    sc2, sh2 = _bn_scale_shift(st2, N * Ho * Wo, gamma2, beta2)

    NB = 8 if N % 8 == 0 else 1
    out = pl.pallas_call(
        _affine_kernel,
        out_shape=jax.ShapeDtypeStruct((N, Ho * Wo, Cout), jnp.float32),
        grid=(N // NB,),
        in_specs=[pl.BlockSpec((NB, Ho * Wo, Cout), lambda n: (n, 0, 0)),
                  pl.BlockSpec((1, Cout), lambda n: (0, 0)),
                  pl.BlockSpec((1, Cout), lambda n: (0, 0))],
        out_specs=pl.BlockSpec((NB, Ho * Wo, Cout), lambda n: (n, 0, 0)),
        compiler_params=parallel,
    )(y2, sc2, sh2)

    return jnp.transpose(out.reshape(N, Ho, Wo, Cout), (0, 3, 1, 2))


# speedup vs baseline: 1.8492x; 1.2205x over previous
import jax
import jax.numpy as jnp
from jax.experimental import pallas as pl
from jax.experimental.pallas import tpu as pltpu

_WB = 64


def _paired_conv(x_flat, pe_ref, po_ref, w_ref, K, C, H):
    RB = x_flat.shape[0] // _WB - 0
    nrows = pe_ref.shape[0] * 2
    M = (H // 2) * _WB
    for kw in range(K):
        seg = x_flat[kw:kw + nrows, :].reshape(nrows // (2 * _WB), 2, _WB, C)
        pe_ref[:, kw * C:(kw + 1) * C] = seg[:, 0].reshape(-1, C)
        po_ref[:, kw * C:(kw + 1) * C] = seg[:, 1].reshape(-1, C)
    acc = jnp.zeros((M, 2 * C), jnp.float32)
    for j in range(K + 1):
        src = pe_ref if j % 2 == 0 else po_ref
        s = (j // 2) * _WB
        acc += jnp.dot(src[s:s + M, :], w_ref[j],
                       preferred_element_type=jnp.float32)
    return acc


def _conv1_kernel(H, W, C, K):
    def body(x_ref, w_ref, b_ref, y_ref, st_ref, pe_ref, po_ref):
        acc = _paired_conv(x_ref[0], pe_ref, po_ref, w_ref, K, C, H)
        acc = jnp.maximum(acc + b_ref[...], 0.0)
        a3 = acc.reshape(H // 2, _WB, 2 * C)
        msk = jax.lax.broadcasted_iota(jnp.int32, a3.shape, 1) < W
        acc = jnp.where(msk, a3, 0.0).reshape(acc.shape)
        y_ref[0] = acc.astype(jnp.bfloat16)
        st_ref[0] = jnp.concatenate(
            [jnp.sum(acc, axis=0, keepdims=True),
             jnp.sum(acc * acc, axis=0, keepdims=True)], axis=0)

    return body


def _conv2_pool_kernel(H, W, C, K):
    PAD = K // 2
    Ho, Wo = H // 2, W // 2
    M = Ho * _WB

    def body(y1_ref, sc_ref, sh_ref, w_ref, b_ref, y2_ref, st_ref,
             xf_ref, pe_ref, po_ref):
        z = y1_ref[0].astype(jnp.float32) * sc_ref[...] + sh_ref[...]
        z3 = z.reshape(Ho, _WB, 2 * C)
        msk = jax.lax.broadcasted_iota(jnp.int32, z3.shape, 1) < W
        z = jnp.where(msk, z3, 0.0).astype(jnp.bfloat16)
        ze = z[:, :, :C].reshape(Ho, 1, _WB, C)
        zo = z[:, :, C:].reshape(Ho, 1, _WB, C)
        zf = jnp.concatenate([ze, zo], axis=1).reshape(H * _WB, C)
        xf_ref[...] = jnp.zeros_like(xf_ref)
        off = PAD * _WB + PAD
        xf_ref[off:off + H * _WB, :] = zf

        acc = _paired_conv(xf_ref[...], pe_ref, po_ref, w_ref, K, C, H)
        acc = jnp.maximum(acc + b_ref[...], 0.0)
        ph = jnp.maximum(acc[:, :C], acc[:, C:])
        pw = ph.reshape(Ho, _WB // 2, 2, C)
        pooled = jnp.maximum(pw[:, :, 0], pw[:, :, 1])
        pmsk = jax.lax.broadcasted_iota(jnp.int32, pooled.shape, 1) < Wo
        pooled = jnp.where(pmsk, pooled, 0.0).reshape(Ho * (_WB // 2), C)
        y2_ref[0] = pooled.astype(jnp.bfloat16)
        st_ref[0] = jnp.concatenate(
            [jnp.sum(pooled, axis=0, keepdims=True),
             jnp.sum(pooled * pooled, axis=0, keepdims=True)], axis=0)

    return body


def _affine_kernel(y_ref, sc_ref, sh_ref, o_ref):
    o_ref[...] = y_ref[...].astype(jnp.float32) * sc_ref[...] + sh_ref[...]


def _pair_weights(wr, K, C):
    z = jnp.zeros_like(wr[:1])
    left = jnp.concatenate([wr, z], axis=0)
    right = jnp.concatenate([z, wr], axis=0)
    return jnp.concatenate([left, right], axis=2)


def _bn_scale_shift(stats, count, gamma, beta, C, eps=1e-5):
    s = jnp.sum(stats, axis=0)
    if s.shape[-1] == 2 * C:
        s = s[:, :C] + s[:, C:]
    mean = s[0] / count
    var = jnp.maximum(s[1] / count - mean * mean, 0.0)
    scale = gamma * jax.lax.rsqrt(var + eps)
    shift = beta - mean * scale
    return (scale.reshape(1, -1).astype(jnp.float32),
            shift.reshape(1, -1).astype(jnp.float32))


def kernel(x, w1, b1, gamma1, beta1, w2, b2, gamma2, beta2):
    N, Cin, H, W = x.shape
    K = w1.shape[0]
    C = w1.shape[-1]
    PAD = K // 2
    Hp = H + 2 * PAD
    Ho, Wo = H // 2, W // 2
    NR = Hp * _WB + 8
    HPE = ((Hp + 1) // 2) * _WB
    M = Ho * _WB

    xh = jnp.transpose(x, (0, 2, 3, 1)).astype(jnp.bfloat16)
    xp = jnp.pad(xh, ((0, 0), (PAD, PAD), (PAD, _WB - W - PAD), (0, 0)))
    xf = jnp.pad(xp.reshape(N, Hp * _WB, C), ((0, 0), (0, NR - Hp * _WB),
                                              (0, 0)))
    w1p = _pair_weights(w1.reshape(K, K * Cin, C).astype(jnp.bfloat16), K, C)
    w2p = _pair_weights(w2.reshape(K, K * C, C).astype(jnp.bfloat16), K, C)
    b1p = jnp.tile(b1.reshape(1, C), (1, 2)).astype(jnp.float32)
    b2p = jnp.tile(b2.reshape(1, C), (1, 2)).astype(jnp.float32)

    parallel = pltpu.CompilerParams(dimension_semantics=("parallel",))

    y1, st1 = pl.pallas_call(
        _conv1_kernel(H, W, C, K),
        out_shape=(jax.ShapeDtypeStruct((N, M, 2 * C), jnp.bfloat16),
                   jax.ShapeDtypeStruct((N, 2, 2 * C), jnp.float32)),
        grid=(N,),
        in_specs=[pl.BlockSpec((1, NR, C), lambda n: (n, 0, 0)),
                  pl.BlockSpec((K + 1, K * Cin, 2 * C), lambda n: (0, 0, 0)),
                  pl.BlockSpec((1, 2 * C), lambda n: (0, 0))],
        out_specs=(pl.BlockSpec((1, M, 2 * C), lambda n: (n, 0, 0)),
                   pl.BlockSpec((1, 2, 2 * C), lambda n: (n, 0, 0))),
        scratch_shapes=[pltpu.VMEM((HPE, K * Cin), jnp.bfloat16),
                        pltpu.VMEM((HPE, K * Cin), jnp.bfloat16)],
        compiler_params=parallel,
    )(xf, w1p, b1p)
    sc1, sh1 = _bn_scale_shift(st1, N * H * W, gamma1, beta1, C)
    sc1p = jnp.tile(sc1, (1, 2))
    sh1p = jnp.tile(sh1, (1, 2))

    y2, st2 = pl.pallas_call(
        _conv2_pool_kernel(H, W, C, K),
        out_shape=(jax.ShapeDtypeStruct((N, Ho * (_WB // 2), C),
                                        jnp.bfloat16),
                   jax.ShapeDtypeStruct((N, 2, C), jnp.float32)),
        grid=(N,),
        in_specs=[pl.BlockSpec((1, M, 2 * C), lambda n: (n, 0, 0)),
                  pl.BlockSpec((1, 2 * C), lambda n: (0, 0)),
                  pl.BlockSpec((1, 2 * C), lambda n: (0, 0)),
                  pl.BlockSpec((K + 1, K * C, 2 * C), lambda n: (0, 0, 0)),
                  pl.BlockSpec((1, 2 * C), lambda n: (0, 0))],
        out_specs=(pl.BlockSpec((1, Ho * (_WB // 2), C), lambda n: (n, 0, 0)),
                   pl.BlockSpec((1, 2, C), lambda n: (n, 0, 0))),
        scratch_shapes=[pltpu.VMEM((NR, C), jnp.bfloat16),
                        pltpu.VMEM((HPE, K * C), jnp.bfloat16),
                        pltpu.VMEM((HPE, K * C), jnp.bfloat16)],
        compiler_params=parallel,
    )(y1, sc1p, sh1p, w2p, b2p)
    sc2, sh2 = _bn_scale_shift(st2, N * Ho * Wo, gamma2, beta2, C)

    NB = 8 if N % 8 == 0 else 1
    WoB = _WB // 2
    out = pl.pallas_call(
        _affine_kernel,
        out_shape=jax.ShapeDtypeStruct((N, Ho * WoB, C), jnp.float32),
        grid=(N // NB,),
        in_specs=[pl.BlockSpec((NB, Ho * WoB, C), lambda n: (n, 0, 0)),
                  pl.BlockSpec((1, C), lambda n: (0, 0)),
                  pl.BlockSpec((1, C), lambda n: (0, 0))],
        out_specs=pl.BlockSpec((NB, Ho * WoB, C), lambda n: (n, 0, 0)),
        compiler_params=parallel,
    )(y2, sc2, sh2)

    out = out.reshape(N, Ho, WoB, C)[:, :, :Wo, :]
    return jnp.transpose(out, (0, 3, 1, 2))
